# Initial kernel scaffold; baseline (speedup 1.0000x reference)
import jax
import jax.numpy as jnp
from jax.experimental import pallas as pl


def kernel(protein_node_feat, protein_edge_index, node_feat, edge_index, edge_attr, batch, params):
    raise NotImplementedError("write your pallas kernel here")


# speedup vs baseline: 5.2941x; 5.2941x over previous
import functools

import jax
import jax.numpy as jnp
from jax import lax
from jax.experimental import pallas as pl
from jax.experimental.pallas import tpu as pltpu
from jax.experimental.pallas import tpu_sc as plsc

N_P = 10000
E_P = 160000
N_M = 256
E_M = 512
B = 8
D = 128
NCORE = 2
NSUB = 16
NTILE = NCORE * NSUB
EPT = E_P // NTILE
ROWS_PS = N_P // NSUB
CH = 40
NCH = EPT // CH
NPT = N_M // NTILE

_SC_PARAMS = pltpu.CompilerParams(needs_layout_passes=False,
                                  use_tc_tiling_on_sc=False)


@functools.cache
def _sc_mesh():
    return plsc.VectorSubcoreMesh(core_axis_name="c", subcore_axis_name="s",
                                  num_cores=NCORE, num_subcores=NSUB)


def _wid():
    return lax.axis_index("s") * NCORE + lax.axis_index("c")


def _sc_prep_body(pdst, msrc, mdst, nf_idx, emb, zeros,
                  degp, c_out, nf_out,
                  degbuf, dstbuf, idxbuf, rowsbuf, cbuf, msbuf, mdbuf, nfbuf,
                  sem):
    wid = _wid()
    ones = jnp.ones((16,), jnp.float32)

    pltpu.sync_copy(zeros.at[pl.ds(0, N_P)], degbuf)
    pltpu.sync_copy(pdst.at[pl.ds(wid * EPT, EPT)], dstbuf.at[pl.ds(0, EPT)])

    def deg_step(i, _):
        idx = dstbuf[pl.ds(i * 16, 16)]
        plsc.addupdate_scatter(degbuf, [idx], ones)
        return 0

    nfull = EPT // 16
    lax.fori_loop(0, nfull, deg_step, 0)
    lane = lax.iota(jnp.int32, 16)
    tmask = lane < (EPT - nfull * 16)
    tidx = jnp.where(tmask, dstbuf[pl.ds(nfull * 16, 16)], 0)
    plsc.addupdate_scatter(degbuf, [tidx], ones, mask=tmask)
    pltpu.sync_copy(degbuf, degp.at[wid])

    @pl.when(wid == 0)
    def _():
        pltpu.sync_copy(zeros, cbuf)
        pltpu.sync_copy(msrc, msbuf)
        pltpu.sync_copy(mdst, mdbuf)

        def mol_step(i, _):
            sv = msbuf[pl.ds(i * 16, 16)]
            dv = mdbuf[pl.ds(i * 16, 16)]
            plsc.addupdate_scatter(cbuf, [dv * N_M + sv], ones)
            return 0

        lax.fori_loop(0, E_M // 16, mol_step, 0)
        pltpu.sync_copy(cbuf, c_out)

    pltpu.sync_copy(nf_idx.at[pl.ds(wid * NPT * 9, NPT * 9)], idxbuf)
    pltpu.async_copy(emb.at[idxbuf], rowsbuf, sem).wait()
    ninth = jnp.full((16,), 1.0 / 9.0, jnp.float32)

    def nf_step(n, _):
        for cc in range(8):
            acc = rowsbuf[n * 9, pl.ds(cc * 16, 16)]
            for j in range(1, 9):
                acc = acc + rowsbuf[n * 9 + j, pl.ds(cc * 16, 16)]
            nfbuf[n, pl.ds(cc * 16, 16)] = acc * ninth
        return 0

    lax.fori_loop(0, NPT, nf_step, 0)
    pltpu.sync_copy(nfbuf, nf_out.at[pl.ds(wid * NPT, NPT)])


@jax.jit
def _sc_prep(pdst, msrc, mdst, nf_idx, emb, zeros):
    return pl.kernel(
        _sc_prep_body,
        out_type=(
            jax.ShapeDtypeStruct((NTILE, N_P), jnp.float32),
            jax.ShapeDtypeStruct((N_M * N_M,), jnp.float32),
            jax.ShapeDtypeStruct((N_M, D), jnp.float32),
        ),
        mesh=_sc_mesh(),
        compiler_params=_SC_PARAMS,
        scratch_types=(
            pltpu.VMEM((N_P,), jnp.float32),
            pltpu.VMEM((EPT + 16,), jnp.int32),
            pltpu.VMEM((NPT * 9,), jnp.int32),
            pltpu.VMEM((NPT * 9, D), jnp.float32),
            pltpu.VMEM((N_M * N_M,), jnp.float32),
            pltpu.VMEM((E_M,), jnp.int32),
            pltpu.VMEM((E_M,), jnp.int32),
            pltpu.VMEM((NPT, D), jnp.float32),
            pltpu.SemaphoreType.DMA,
        ),
    )(pdst, msrc, mdst, nf_idx, emb, zeros)


def _sc_scatter_body(hs, psrc, pdst, zrows,
                     out0, out1,
                     sidx, didx, rows, shared, sem):
    c = lax.axis_index("c")
    s = lax.axis_index("s")
    wid = s * NCORE + c
    pltpu.sync_copy(zrows, shared.at[pl.ds(s * ROWS_PS, ROWS_PS)])
    plsc.subcore_barrier()
    base = wid * EPT

    def step(i, _):
        pltpu.sync_copy(psrc.at[pl.ds(base + i * CH, CH)], sidx)
        pltpu.sync_copy(pdst.at[pl.ds(base + i * CH, CH)], didx)
        pltpu.async_copy(hs.at[sidx], rows, sem).wait()
        pltpu.sync_copy(rows, shared.at[didx], add=True)
        return 0

    lax.fori_loop(0, NCH, step, 0)
    plsc.subcore_barrier()
    sl = pl.ds(s * ROWS_PS, ROWS_PS)

    @pl.when(c == 0)
    def _():
        pltpu.sync_copy(shared.at[sl], out0.at[sl])

    @pl.when(c == 1)
    def _():
        pltpu.sync_copy(shared.at[sl], out1.at[sl])


@jax.jit
def _sc_scatter(hs, psrc, pdst, zrows):
    return pl.kernel(
        _sc_scatter_body,
        out_type=(
            jax.ShapeDtypeStruct((N_P, D), jnp.float32),
            jax.ShapeDtypeStruct((N_P, D), jnp.float32),
        ),
        mesh=_sc_mesh(),
        compiler_params=_SC_PARAMS,
        scratch_types=(
            pltpu.VMEM((CH,), jnp.int32),
            pltpu.VMEM((CH,), jnp.int32),
            pltpu.VMEM((CH, D), jnp.float32),
            pltpu.VMEM_SHARED((N_P, D), jnp.float32),
            pltpu.SemaphoreType.DMA,
        ),
    )(hs, psrc, pdst, zrows)


def _sc_adj_body(s0, psrc, pdst,
                 out,
                 s0v, srcb, dstb, outb):
    wid = _wid()
    pltpu.sync_copy(s0, s0v)
    pltpu.sync_copy(psrc.at[pl.ds(wid * EPT, EPT)], srcb.at[pl.ds(0, EPT)])
    pltpu.sync_copy(pdst.at[pl.ds(wid * EPT, EPT)], dstb.at[pl.ds(0, EPT)])
    zero = jnp.zeros((16,), jnp.float32)
    init = tuple(zero for _ in range(24))

    def step(i, accs):
        sv = srcb[pl.ds(i * 16, 16)] * B
        dv = dstb[pl.ds(i * 16, 16)] * B
        accs = list(accs)
        for b in range(B):
            gs = plsc.load_gather(s0v, [sv + b])
            gd = plsc.load_gather(s0v, [dv + b])
            accs[b] = accs[b] + gs * gd
            accs[8 + b] = accs[8 + b] + gs
            accs[16 + b] = accs[16 + b] + gd
        return tuple(accs)

    nfull = EPT // 16
    accs = lax.fori_loop(0, nfull, step, init)
    lane = lax.iota(jnp.int32, 16)
    tmask = lane < (EPT - nfull * 16)
    sv = jnp.where(tmask, srcb[pl.ds(nfull * 16, 16)], 0) * B
    dv = jnp.where(tmask, dstb[pl.ds(nfull * 16, 16)], 0) * B
    accs = list(accs)
    for b in range(B):
        gs = plsc.load_gather(s0v, [sv + b], mask=tmask)
        gd = plsc.load_gather(s0v, [dv + b], mask=tmask)
        gs = jnp.where(tmask, gs, 0.0)
        gd = jnp.where(tmask, gd, 0.0)
        accs[b] = accs[b] + gs * gd
        accs[8 + b] = accs[8 + b] + gs
        accs[16 + b] = accs[16 + b] + gd
    for t in range(24):
        outb[t, :] = accs[t]
    pltpu.sync_copy(outb, out.at[wid])


@jax.jit
def _sc_adj(s0, psrc, pdst):
    return pl.kernel(
        _sc_adj_body,
        out_type=jax.ShapeDtypeStruct((NTILE, 24, 16), jnp.float32),
        mesh=_sc_mesh(),
        compiler_params=_SC_PARAMS,
        scratch_types=(
            pltpu.VMEM((N_P * B,), jnp.float32),
            pltpu.VMEM((EPT + 16,), jnp.int32),
            pltpu.VMEM((EPT + 16,), jnp.int32),
            pltpu.VMEM((24, 16), jnp.float32),
        ),
    )(s0, psrc, pdst)


_RB = 2000
_NG = N_P // _RB


def _k_deg_body(degp, dinv, ideg):
    deg = jnp.sum(degp[...], axis=0, keepdims=True) + 1.0
    dinv[...] = lax.rsqrt(deg)
    ideg[...] = 1.0 / deg


@jax.jit
def _k_deg(degp):
    return pl.pallas_call(
        _k_deg_body,
        out_shape=(
            jax.ShapeDtypeStruct((1, N_P), jnp.float32),
            jax.ShapeDtypeStruct((1, N_P), jnp.float32),
        ),
    )(degp)


def _k_mm0_body(x, w, dinv, h, hs):
    hh = jnp.dot(x[...], w[...], preferred_element_type=jnp.float32)
    h[...] = hh
    hs[...] = hh * dinv[...]


@jax.jit
def _k_mm0(x, w, dinv):
    return pl.pallas_call(
        _k_mm0_body,
        grid=(_NG,),
        in_specs=[
            pl.BlockSpec((_RB, D), lambda i: (i, 0)),
            pl.BlockSpec((D, D), lambda i: (0, 0)),
            pl.BlockSpec((_RB, 1), lambda i: (i, 0)),
        ],
        out_specs=(
            pl.BlockSpec((_RB, D), lambda i: (i, 0)),
            pl.BlockSpec((_RB, D), lambda i: (i, 0)),
        ),
        out_shape=(
            jax.ShapeDtypeStruct((N_P, D), jnp.float32),
            jax.ShapeDtypeStruct((N_P, D), jnp.float32),
        ),
    )(x, w, dinv)


def _k_combine_body(p0, p1, h, dinv, ideg, bvec, y, stats, acc):
    yv = (p0[...] + p1[...]) * dinv[...] + h[...] * ideg[...] + bvec[...]
    y[...] = yv

    @pl.when(pl.program_id(0) == 0)
    def _():
        acc[...] = jnp.zeros_like(acc)

    acc[0:1, :] += jnp.sum(yv, axis=0, keepdims=True)
    acc[1:2, :] += jnp.sum(yv * yv, axis=0, keepdims=True)

    @pl.when(pl.program_id(0) == _NG - 1)
    def _():
        stats[...] = acc[...]


@jax.jit
def _k_combine(p0, p1, h, dinv, ideg, bvec):
    return pl.pallas_call(
        _k_combine_body,
        grid=(_NG,),
        in_specs=[
            pl.BlockSpec((_RB, D), lambda i: (i, 0)),
            pl.BlockSpec((_RB, D), lambda i: (i, 0)),
            pl.BlockSpec((_RB, D), lambda i: (i, 0)),
            pl.BlockSpec((_RB, 1), lambda i: (i, 0)),
            pl.BlockSpec((_RB, 1), lambda i: (i, 0)),
            pl.BlockSpec((1, D), lambda i: (0, 0)),
        ],
        out_specs=(
            pl.BlockSpec((_RB, D), lambda i: (i, 0)),
            pl.BlockSpec((2, D), lambda i: (0, 0)),
        ),
        out_shape=(
            jax.ShapeDtypeStruct((N_P, D), jnp.float32),
            jax.ShapeDtypeStruct((2, D), jnp.float32),
        ),
        scratch_shapes=[pltpu.VMEM((2, D), jnp.float32)],
    )(p0, p1, h, dinv, ideg, bvec)


def _bn_from_stats(yv, stats, g, be, n):
    mu = stats[0:1, :] * (1.0 / n)
    var = stats[1:2, :] * (1.0 / n) - mu * mu
    return (yv - mu) * lax.rsqrt(var + 1e-5) * g + be


def _k_mm_body(y, stats, g, be, w, dinv, h, hs):
    xv = jax.nn.relu(_bn_from_stats(y[...], stats[...], g[...], be[...], N_P))
    hh = jnp.dot(xv, w[...], preferred_element_type=jnp.float32)
    h[...] = hh
    hs[...] = hh * dinv[...]


@jax.jit
def _k_mm(y, stats, g, be, w, dinv):
    return pl.pallas_call(
        _k_mm_body,
        grid=(_NG,),
        in_specs=[
            pl.BlockSpec((_RB, D), lambda i: (i, 0)),
            pl.BlockSpec((2, D), lambda i: (0, 0)),
            pl.BlockSpec((1, D), lambda i: (0, 0)),
            pl.BlockSpec((1, D), lambda i: (0, 0)),
            pl.BlockSpec((D, D), lambda i: (0, 0)),
            pl.BlockSpec((_RB, 1), lambda i: (i, 0)),
        ],
        out_specs=(
            pl.BlockSpec((_RB, D), lambda i: (i, 0)),
            pl.BlockSpec((_RB, D), lambda i: (i, 0)),
        ),
        out_shape=(
            jax.ShapeDtypeStruct((N_P, D), jnp.float32),
            jax.ShapeDtypeStruct((N_P, D), jnp.float32),
        ),
    )(y, stats, g, be, w, dinv)


def _k_mol_body(nf, cmat, batch, w0, b0, g0, be0, w1, b1, g1, be1,
                w2, b2, g2, be2, mol):
    c = cmat[...]
    deg = jnp.sum(c, axis=1, keepdims=True) + 1.0
    dinv = lax.rsqrt(deg)
    ideg = 1.0 / deg
    x = nf[...]
    for (w, bb, g, be, last) in (
        (w0, b0, g0, be0, False),
        (w1, b1, g1, be1, False),
        (w2, b2, g2, be2, True),
    ):
        h = jnp.dot(x, w[...], preferred_element_type=jnp.float32)
        agg = dinv * jnp.dot(c, dinv * h, preferred_element_type=jnp.float32)
        yv = agg + ideg * h + bb[...]
        mu = jnp.mean(yv, axis=0, keepdims=True)
        var = jnp.mean(yv * yv, axis=0, keepdims=True) - mu * mu
        x = (yv - mu) * lax.rsqrt(var + 1e-5) * g[...] + be[...]
        if not last:
            x = jax.nn.relu(x)
    bb2 = batch[...]
    b2 = bb2 - bb2[0:1, 0:1]
    gid = lax.broadcasted_iota(jnp.int32, (B, N_M), 0)
    maskf = jnp.where(gid == b2, 1.0, 0.0)
    cnt = jnp.maximum(jnp.sum(maskf, axis=1, keepdims=True), 1.0)
    mol[...] = jnp.dot(maskf, x, preferred_element_type=jnp.float32) / cnt


@jax.jit
def _k_mol(nf, cmat, batch, *mw):
    return pl.pallas_call(
        _k_mol_body,
        out_shape=jax.ShapeDtypeStruct((B, D), jnp.float32),
    )(nf, cmat, batch, *mw)


def _k_cluster_body(y, stats, g, be, c1wt, c1wb, c1b, cw, cb, c2w, c2b, mol,
                    s0_out, pos_raw, graph_sum, pos_acc, graph_acc):
    pemb = _bn_from_stats(y[...], stats[...], g[...], be[...], N_P)
    q = jnp.dot(pemb, c1wt[...], preferred_element_type=jnp.float32) + c1b[...]
    r = jnp.dot(mol[...], c1wb[...], preferred_element_type=jnp.float32)
    w2d = c2w[:, 0:1] - c2w[:, 1:2]
    b2d = c2b[0:1, 0:1] - c2b[0:1, 1:2]

    @pl.when(pl.program_id(0) == 0)
    def _():
        pos_acc[...] = jnp.zeros_like(pos_acc)
        graph_acc[...] = jnp.zeros_like(graph_acc)

    graph_acc[...] += jnp.sum(pemb, axis=0, keepdims=True)
    for b in range(B):
        t1 = jax.nn.relu(q + r[b:b + 1, :])
        t2 = jax.nn.relu(jnp.dot(t1, cw[...], preferred_element_type=jnp.float32) + cb[...])
        dl = jnp.dot(t2, w2d, preferred_element_type=jnp.float32) + b2d
        s0 = jax.nn.sigmoid(dl)
        s0_out[:, b:b + 1] = s0
        pos_acc[b:b + 1, :] += lax.dot_general(
            s0, pemb, (((0,), (0,)), ((), ())),
            preferred_element_type=jnp.float32)

    @pl.when(pl.program_id(0) == _NG - 1)
    def _():
        pos_raw[...] = pos_acc[...]
        graph_sum[...] = graph_acc[...]


@jax.jit
def _k_cluster(y, stats, g, be, c1wt, c1wb, c1b, cw, cb, c2w, c2b, mol):
    return pl.pallas_call(
        _k_cluster_body,
        grid=(_NG,),
        in_specs=[
            pl.BlockSpec((_RB, D), lambda i: (i, 0)),
            pl.BlockSpec((2, D), lambda i: (0, 0)),
            pl.BlockSpec((1, D), lambda i: (0, 0)),
            pl.BlockSpec((1, D), lambda i: (0, 0)),
            pl.BlockSpec((D, D), lambda i: (0, 0)),
            pl.BlockSpec((D, D), lambda i: (0, 0)),
            pl.BlockSpec((1, D), lambda i: (0, 0)),
            pl.BlockSpec((D, D), lambda i: (0, 0)),
            pl.BlockSpec((1, D), lambda i: (0, 0)),
            pl.BlockSpec((D, 2), lambda i: (0, 0)),
            pl.BlockSpec((1, 2), lambda i: (0, 0)),
            pl.BlockSpec((B, D), lambda i: (0, 0)),
        ],
        out_specs=(
            pl.BlockSpec((_RB, B), lambda i: (i, 0)),
            pl.BlockSpec((B, D), lambda i: (0, 0)),
            pl.BlockSpec((1, D), lambda i: (0, 0)),
        ),
        out_shape=(
            jax.ShapeDtypeStruct((N_P, B), jnp.float32),
            jax.ShapeDtypeStruct((B, D), jnp.float32),
            jax.ShapeDtypeStruct((1, D), jnp.float32),
        ),
        scratch_shapes=[
            pltpu.VMEM((B, D), jnp.float32),
            pltpu.VMEM((1, D), jnp.float32),
        ],
    )(y, stats, g, be, c1wt, c1wb, c1b, cw, cb, c2w, c2b, mol)


def _k_final_body(pos_raw, graph_sum, mol, adjsums,
                  f1w, f1b, f2w, f2b, l1w, l1b, l2w, l2b, l3w, l3b,
                  pred, pos_emb, graph_emb, pen):
    pos = pos_raw[...] * (1.0 / N_P)
    pos_emb[...] = pos
    graph_emb[...] = jnp.broadcast_to(graph_sum[...] * (1.0 / N_P), (B, D))
    lane = jnp.sum(adjsums[...], axis=1, keepdims=True)
    colmod = lax.broadcasted_iota(jnp.int32, (24, NTILE * 24), 1) % 24
    rowi = lax.broadcasted_iota(jnp.int32, (24, NTILE * 24), 0)
    sel = jnp.where(colmod == rowi, 1.0, 0.0)
    terms = jnp.dot(sel, lane, preferred_element_type=jnp.float32)
    ssd = terms[0:B, :]
    ss = terms[B:2 * B, :]
    sd = terms[2 * B:3 * B, :]
    n00 = ssd
    n01 = ss - ssd
    n10 = sd - ssd
    n11 = float(E_P) - ss - sd + ssd
    l0 = jnp.abs(n00) + jnp.abs(n01)
    l1 = jnp.abs(n10) + jnp.abs(n11)
    d0 = n00 / jnp.maximum(l0, 1e-5)
    d1 = n11 / jnp.maximum(l1, 1e-5)
    pen[...] = (jnp.sum((d0 - 1.0) ** 2) + jnp.sum((d1 - 1.0) ** 2)) * (
        1.0 / (2 * B)) * jnp.ones((1, 1), jnp.float32)
    o = jax.nn.relu(jnp.dot(pos, f1w[...], preferred_element_type=jnp.float32) + f1b[...])
    o = jnp.dot(o, f2w[...], preferred_element_type=jnp.float32) + f2b[...]
    z = jnp.concatenate([o, mol[...]], axis=1)
    z = jax.nn.relu(jnp.dot(z, l1w[...], preferred_element_type=jnp.float32) + l1b[...])
    z = jax.nn.relu(jnp.dot(z, l2w[...], preferred_element_type=jnp.float32) + l2b[...])
    pred[...] = jnp.dot(z, l3w[...], preferred_element_type=jnp.float32) + l3b[...]


@jax.jit
def _k_final(pos_raw, graph_sum, mol, adjsums, *ws):
    return pl.pallas_call(
        _k_final_body,
        out_shape=(
            jax.ShapeDtypeStruct((B, 1), jnp.float32),
            jax.ShapeDtypeStruct((B, D), jnp.float32),
            jax.ShapeDtypeStruct((B, D), jnp.float32),
            jax.ShapeDtypeStruct((1, 1), jnp.float32),
        ),
    )(pos_raw, graph_sum, mol, adjsums, *ws)


def kernel(protein_node_feat, protein_edge_index, node_feat, edge_index,
           edge_attr, batch, params):
    p = params
    psrc = protein_edge_index[0]
    pdst = protein_edge_index[1]
    msrc = edge_index[0]
    mdst = edge_index[1]
    nf_idx = node_feat.reshape(-1)
    zeros = jnp.zeros((N_M * N_M,), jnp.float32)
    zrows = jnp.zeros((ROWS_PS, D), jnp.float32)

    degp, cflat, nf = _sc_prep(pdst, msrc, mdst, nf_idx, p['emb'], zeros)
    dinv_r, ideg_r = _k_deg(degp)
    dinv = dinv_r.reshape(N_P, 1)
    ideg = ideg_r.reshape(N_P, 1)

    h, hs = _k_mm0(protein_node_feat, p['pW0'], dinv)
    y = stats = None
    for l in range(4):
        p0, p1 = _sc_scatter(hs, psrc, pdst, zrows)
        y, stats = _k_combine(p0, p1, h, dinv, ideg,
                              p['pb%d' % l].reshape(1, D))
        if l < 3:
            h, hs = _k_mm(y, stats, p['pg%d' % l].reshape(1, D),
                          p['pbe%d' % l].reshape(1, D), p['pW%d' % (l + 1)],
                          dinv)

    mol = _k_mol(nf, cflat.reshape(N_M, N_M), batch.reshape(1, N_M),
                 p['mW0'], p['mb0'].reshape(1, D), p['mg0'].reshape(1, D),
                 p['mbe0'].reshape(1, D),
                 p['mW1'], p['mb1'].reshape(1, D), p['mg1'].reshape(1, D),
                 p['mbe1'].reshape(1, D),
                 p['mW2'], p['mb2'].reshape(1, D), p['mg2'].reshape(1, D),
                 p['mbe2'].reshape(1, D))

    s0, pos_raw, graph_sum = _k_cluster(
        y, stats, p['pg3'].reshape(1, D), p['pbe3'].reshape(1, D),
        p['c1W'][0:D, :], p['c1W'][D:2 * D, :], p['c1b'].reshape(1, D),
        p['cW'], p['cb'].reshape(1, D), p['c2W'], p['c2b'].reshape(1, 2),
        mol)

    adjsums = _sc_adj(s0.reshape(-1), psrc, pdst).reshape(NTILE * 24, 16)

    pred, pos_emb, graph_emb, pen = _k_final(
        pos_raw, graph_sum, mol, adjsums,
        p['f1W'], p['f1b'].reshape(1, D), p['f2W'], p['f2b'].reshape(1, D),
        p['l1W'], p['l1b'].reshape(1, 2 * D), p['l2W'],
        p['l2b'].reshape(1, 2 * D), p['l3W'], p['l3b'].reshape(1, 1))
    return pred, pos_emb, graph_emb, pen.reshape(())
